# Initial kernel scaffold; baseline (speedup 1.0000x reference)
import jax
import jax.numpy as jnp
from jax.experimental import pallas as pl


def kernel(coords, features, W_inv0, b_inv0, W_inv1, b_inv1, W_feat0, b_feat0, W_feat1, b_feat1, W_sh0, b_sh0, W_sh1, b_sh1, W_g, b_g):
    raise NotImplementedError("write your pallas kernel here")


# speedup vs baseline: 35.9854x; 35.9854x over previous
import functools

import jax
import jax.numpy as jnp
from jax.experimental import pallas as pl

KNN = 16
BIG = 3.0e38


def _gelu(x):
    return 0.5 * x * (1.0 + jax.lax.erf(x * 0.7071067811865476))


def _dotT(w, x):
    return jax.lax.dot_general(w, x, (((0,), (0,)), ((), ())),
                               preferred_element_type=jnp.float32)


def _rowsumT(x):
    ones_row = jnp.ones((1, x.shape[1]), jnp.float32)
    return jax.lax.dot_general(ones_row, x, (((1,), (1,)), ((), ())),
                               preferred_element_type=jnp.float32,
                               precision=jax.lax.Precision.HIGHEST)


def _jacobi3(Am, R):
    one = jnp.ones((1, R), jnp.float32)
    zero = jnp.zeros((1, R), jnp.float32)
    V = [[one, zero, zero], [zero, one, zero], [zero, zero, one]]
    for _ in range(5):
        for (p, q) in ((0, 1), (0, 2), (1, 2)):
            app = Am[p][p]
            aqq = Am[q][q]
            apq = Am[p][q]
            small = jnp.abs(apq) < 1e-32
            apq_s = jnp.where(small, 1.0, apq)
            tau = (aqq - app) * 0.5 / apq_s
            sgn = jnp.where(tau >= 0.0, 1.0, -1.0)
            t = sgn / (jnp.abs(tau) + jnp.sqrt(1.0 + tau * tau))
            t = jnp.where(small, 0.0, t)
            c = jax.lax.rsqrt(1.0 + t * t)
            s = t * c
            r = 3 - p - q
            arp = Am[r][p]
            arq = Am[r][q]
            Am[p][p] = app - t * apq
            Am[q][q] = aqq + t * apq
            Am[p][q] = zero
            Am[q][p] = zero
            nrp = c * arp - s * arq
            nrq = s * arp + c * arq
            Am[r][p] = nrp
            Am[p][r] = nrp
            Am[r][q] = nrq
            Am[q][r] = nrq
            for i in range(3):
                vp = V[i][p]
                vq = V[i][q]
                V[i][p] = c * vp - s * vq
                V[i][q] = s * vp + c * vq
    return [Am[0][0], Am[1][1], Am[2][2]], V


def _body(coords_ref, coordsT_ref, featT_ref,
          Wi0_ref, bi0_ref, Wi1_ref, bi1_ref,
          Wf0_ref, bf0_ref, Wf1_ref, bf1_ref,
          Ws0_ref, bs0_ref, Ws1_ref, bs1_ref,
          Wg_ref, bg_ref, out_ref, *, R, N):
    nb = pl.program_id(1)
    ct = coordsT_ref[0]
    cr = coords_ref[0]
    xi = cr[:, 0:1]
    yi = cr[:, 1:2]
    zi = cr[:, 2:3]
    xj = ct[0:1, :]
    yj = ct[1:2, :]
    zj = ct[2:3, :]
    dot = jax.lax.dot_general(cr, ct, (((1,), (0,)), ((), ())),
                              preferred_element_type=jnp.float32)
    sqj = xj * xj + yj * yj + zj * zj
    sqi = xi * xi + yi * yi + zi * zi
    d2 = sqi + sqj - 2.0 * dot
    rows = nb * R + jax.lax.broadcasted_iota(jnp.int32, (R, N), 0)
    cols = jax.lax.broadcasted_iota(jnp.int32, (R, N), 1)
    d2 = jnp.where(rows == cols, BIG, d2)

    def step(_, carry):
        d2w, _t = carry
        m = jnp.min(d2w, axis=1, keepdims=True)
        d2w = jnp.where(d2w <= m, BIG, d2w)
        return d2w, m

    _, t = jax.lax.fori_loop(0, KNN, step,
                             (d2, jnp.zeros((R, 1), jnp.float32)))
    maskb = d2 <= t
    distm = jnp.sqrt(jnp.where(maskb, jnp.maximum(d2, 0.0), 0.0))
    rsum = _rowsumT(distm)

    relx = xj - xi
    rely = yj - yi
    relz = zj - zi
    bx = relx.astype(jnp.bfloat16).astype(jnp.float32)
    by = rely.astype(jnp.bfloat16).astype(jnp.float32)
    bz = relz.astype(jnp.bfloat16).astype(jnp.float32)
    zeros = jnp.zeros((R, N), jnp.float32)
    mbx = jnp.where(maskb, bx, zeros)
    mby = jnp.where(maskb, by, zeros)
    mbz = jnp.where(maskb, bz, zeros)
    invk = 1.0 / float(KNN)
    axx = _rowsumT(mbx * bx) * invk
    ayy = _rowsumT(mby * by) * invk
    azz = _rowsumT(mbz * bz) * invk
    axy = _rowsumT(mbx * by) * invk
    axz = _rowsumT(mbx * bz) * invk
    ayz = _rowsumT(mby * bz) * invk

    ciT = coordsT_ref[0, :, pl.ds(nb * R, R)]
    xT = ciT[0:1]
    yT = ciT[1:2]
    zT = ciT[2:3]

    Am = [[axx, axy, axz], [axy, ayy, ayz], [axz, ayz, azz]]
    (e0, e1, e2), V = _jacobi3(Am, R)

    lmin = jnp.minimum(jnp.minimum(e0, e1), e2)
    lmax = jnp.maximum(jnp.maximum(e0, e1), e2)
    lmid = e0 + e1 + e2 - lmin - lmax
    pick0 = jnp.logical_and(e0 <= e1, e0 <= e2)
    pick1 = jnp.logical_and(jnp.logical_not(pick0), e1 <= e2)
    nx = jnp.where(pick0, V[0][0], jnp.where(pick1, V[0][1], V[0][2]))
    ny = jnp.where(pick0, V[1][0], jnp.where(pick1, V[1][1], V[1][2]))
    nz = jnp.where(pick0, V[2][0], jnp.where(pick1, V[2][1], V[2][2]))

    cenx = jnp.mean(ct[0:1, :], axis=1, keepdims=True)
    ceny = jnp.mean(ct[1:2, :], axis=1, keepdims=True)
    cenz = jnp.mean(ct[2:3, :], axis=1, keepdims=True)
    ox = xT - cenx
    oy = yT - ceny
    oz = zT - cenz
    orient = jnp.where(nx * ox + ny * oy + nz * oz >= 0.0, 1.0, -1.0)
    nx = nx * orient
    ny = ny * orient
    nz = nz * orient
    nrm = jnp.maximum(jnp.sqrt(nx * nx + ny * ny + nz * nz), 1e-6)
    nx = nx / nrm
    ny = ny / nrm
    nz = nz / nrm

    radius = rsum * invk
    cenrad = jnp.sqrt(ox * ox + oy * oy + oz * oz)
    esum = jnp.maximum(lmin + lmid + lmax, 1e-6)
    dom = lmax / esum
    invariants = jnp.concatenate(
        [lmin, lmid, lmax, radius, cenrad, dom], axis=0)

    h = _gelu(_dotT(Wi0_ref[...], invariants) + bi0_ref[...])
    inv_h = _dotT(Wi1_ref[...], h) + bi1_ref[...]
    fT = featT_ref[0]
    h = _gelu(_dotT(Wf0_ref[...], fT) + bf0_ref[...])
    feat_h = _dotT(Wf1_ref[...], h) + bf1_ref[...]
    hcat = jnp.concatenate([inv_h, feat_h], axis=0)
    h = _gelu(_dotT(Ws0_ref[...], hcat) + bs0_ref[...])
    hidden = _dotT(Ws1_ref[...], h) + bs1_ref[...]
    g0 = _dotT(Wg_ref[...], hidden) + bg_ref[...]

    zrow = jnp.zeros((1, R), jnp.float32)
    onesr = jnp.ones((1, R), jnp.float32)
    pd = -(xT * nx + yT * ny + zT * nz)
    outT = jnp.concatenate([
        g0,
        nx, ny, -nz, pd,
        nx, ny, nz, zrow, zrow, zrow,
        xT, yT, zT, onesr,
        zrow], axis=0)
    out_ref[0] = outT


def kernel(coords, features, W_inv0, b_inv0, W_inv1, b_inv1,
           W_feat0, b_feat0, W_feat1, b_feat1,
           W_sh0, b_sh0, W_sh1, b_sh1, W_g, b_g):
    B, N, _ = coords.shape
    R = 256 if N % 256 == 0 else N
    NB = N // R
    coordsT = jnp.transpose(coords, (0, 2, 1))
    featT = jnp.transpose(features, (0, 2, 1))
    col = lambda b: b.reshape(-1, 1)

    grid = (B, NB)
    full = lambda b, nb: (0, 0)
    outT = pl.pallas_call(
        functools.partial(_body, R=R, N=N),
        grid=grid,
        in_specs=[
            pl.BlockSpec((1, R, 3), lambda b, nb: (b, nb, 0)),
            pl.BlockSpec((1, 3, N), lambda b, nb: (b, 0, 0)),
            pl.BlockSpec((1, 64, R), lambda b, nb: (b, 0, nb)),
            pl.BlockSpec((6, 64), full), pl.BlockSpec((64, 1), full),
            pl.BlockSpec((64, 64), full), pl.BlockSpec((64, 1), full),
            pl.BlockSpec((64, 64), full), pl.BlockSpec((64, 1), full),
            pl.BlockSpec((64, 64), full), pl.BlockSpec((64, 1), full),
            pl.BlockSpec((128, 64), full), pl.BlockSpec((64, 1), full),
            pl.BlockSpec((64, 64), full), pl.BlockSpec((64, 1), full),
            pl.BlockSpec((64, 1), full), pl.BlockSpec((1, 1), full),
        ],
        out_specs=pl.BlockSpec((1, 16, R), lambda b, nb: (b, 0, nb)),
        out_shape=jax.ShapeDtypeStruct((B, 16, N), jnp.float32),
    )(coords, coordsT, featT,
      W_inv0, col(b_inv0), W_inv1, col(b_inv1),
      W_feat0, col(b_feat0), W_feat1, col(b_feat1),
      W_sh0, col(b_sh0), W_sh1, col(b_sh1),
      W_g, col(b_g))
    return jnp.transpose(outT, (0, 2, 1))


# speedup vs baseline: 62.4838x; 1.7364x over previous
import functools

import jax
import jax.numpy as jnp
from jax.experimental import pallas as pl

KNN = 16
BIG = 3.0e38


def _gelu(x):
    return 0.5 * x * (1.0 + jax.lax.erf(x * 0.7071067811865476))


def _dotT(w, x):
    return jax.lax.dot_general(w, x, (((0,), (0,)), ((), ())),
                               preferred_element_type=jnp.float32)


def _transpose_small(x, eye):
    return jax.lax.dot_general(x, eye, (((0,), (0,)), ((), ())),
                               preferred_element_type=jnp.float32,
                               precision=jax.lax.Precision.HIGHEST)


def _jacobi3(Am, R):
    one = jnp.ones((1, R), jnp.float32)
    zero = jnp.zeros((1, R), jnp.float32)
    V = [[one, zero, zero], [zero, one, zero], [zero, zero, one]]
    for _ in range(5):
        for (p, q) in ((0, 1), (0, 2), (1, 2)):
            app = Am[p][p]
            aqq = Am[q][q]
            apq = Am[p][q]
            small = jnp.abs(apq) < 1e-32
            apq_s = jnp.where(small, 1.0, apq)
            tau = (aqq - app) * 0.5 / apq_s
            sgn = jnp.where(tau >= 0.0, 1.0, -1.0)
            t = sgn / (jnp.abs(tau) + jnp.sqrt(1.0 + tau * tau))
            t = jnp.where(small, 0.0, t)
            c = jax.lax.rsqrt(1.0 + t * t)
            s = t * c
            r = 3 - p - q
            arp = Am[r][p]
            arq = Am[r][q]
            Am[p][p] = app - t * apq
            Am[q][q] = aqq + t * apq
            Am[p][q] = zero
            Am[q][p] = zero
            nrp = c * arp - s * arq
            nrq = s * arp + c * arq
            Am[r][p] = nrp
            Am[p][r] = nrp
            Am[r][q] = nrq
            Am[q][r] = nrq
            for i in range(3):
                vp = V[i][p]
                vq = V[i][q]
                V[i][p] = c * vp - s * vq
                V[i][q] = s * vp + c * vq
    return [Am[0][0], Am[1][1], Am[2][2]], V


def _body(coords_ref, coordsT_ref, featT_ref, eye_ref,
          Wi0_ref, bi0_ref, Wi1_ref, bi1_ref,
          Wf0_ref, bf0_ref, Wf1_ref, bf1_ref,
          Ws0_ref, bs0_ref, Ws1_ref, bs1_ref,
          Wg_ref, bg_ref, out_ref, *, R, N):
    nb = pl.program_id(1)
    ct = coordsT_ref[0]
    cr = coords_ref[0]
    xi = cr[:, 0:1]
    yi = cr[:, 1:2]
    zi = cr[:, 2:3]
    xj = ct[0:1, :]
    yj = ct[1:2, :]
    zj = ct[2:3, :]
    dot = jax.lax.dot_general(cr, ct, (((1,), (0,)), ((), ())),
                              preferred_element_type=jnp.float32)
    sqj = xj * xj + yj * yj + zj * zj
    sqi = xi * xi + yi * yi + zi * zi
    d2 = sqi + sqj - 2.0 * dot
    rows = nb * R + jax.lax.broadcasted_iota(jnp.int32, (R, N), 0)
    cols = jax.lax.broadcasted_iota(jnp.int32, (R, N), 1)
    d2 = jnp.where(rows == cols, BIG, d2)

    def step(_, carry):
        d2w, _t = carry
        m = jnp.min(d2w, axis=1, keepdims=True)
        d2w = jnp.where(d2w <= m, BIG, d2w)
        return d2w, m

    _, t = jax.lax.fori_loop(0, KNN, step,
                             (d2, jnp.zeros((R, 1), jnp.float32)))
    maskb = d2 <= t
    distm = jnp.sqrt(jnp.where(maskb, jnp.maximum(d2, 0.0), 0.0))

    relx = xj - xi
    rely = yj - yi
    relz = zj - zi
    bx = relx.astype(jnp.bfloat16).astype(jnp.float32)
    by = rely.astype(jnp.bfloat16).astype(jnp.float32)
    bz = relz.astype(jnp.bfloat16).astype(jnp.float32)
    zeros = jnp.zeros((R, N), jnp.float32)
    mbx = jnp.where(maskb, bx, zeros)
    mby = jnp.where(maskb, by, zeros)
    mbz = jnp.where(maskb, bz, zeros)
    invk = 1.0 / float(KNN)

    def rsum1(x):
        return jnp.sum(x, axis=1, keepdims=True)

    scal = jnp.concatenate([
        rsum1(mbx * bx), rsum1(mby * by), rsum1(mbz * bz),
        rsum1(mbx * by), rsum1(mbx * bz), rsum1(mby * bz),
        rsum1(distm), jnp.zeros((R, 1), jnp.float32)], axis=1)
    scalT = _transpose_small(scal, eye_ref[...])
    axx = scalT[0:1] * invk
    ayy = scalT[1:2] * invk
    azz = scalT[2:3] * invk
    axy = scalT[3:4] * invk
    axz = scalT[4:5] * invk
    ayz = scalT[5:6] * invk
    rsum = scalT[6:7]

    ciT = coordsT_ref[0, :, pl.ds(nb * R, R)]
    xT = ciT[0:1]
    yT = ciT[1:2]
    zT = ciT[2:3]

    Am = [[axx, axy, axz], [axy, ayy, ayz], [axz, ayz, azz]]
    (e0, e1, e2), V = _jacobi3(Am, R)

    lmin = jnp.minimum(jnp.minimum(e0, e1), e2)
    lmax = jnp.maximum(jnp.maximum(e0, e1), e2)
    lmid = e0 + e1 + e2 - lmin - lmax
    pick0 = jnp.logical_and(e0 <= e1, e0 <= e2)
    pick1 = jnp.logical_and(jnp.logical_not(pick0), e1 <= e2)
    nx = jnp.where(pick0, V[0][0], jnp.where(pick1, V[0][1], V[0][2]))
    ny = jnp.where(pick0, V[1][0], jnp.where(pick1, V[1][1], V[1][2]))
    nz = jnp.where(pick0, V[2][0], jnp.where(pick1, V[2][1], V[2][2]))

    cenx = jnp.mean(ct[0:1, :], axis=1, keepdims=True)
    ceny = jnp.mean(ct[1:2, :], axis=1, keepdims=True)
    cenz = jnp.mean(ct[2:3, :], axis=1, keepdims=True)
    ox = xT - cenx
    oy = yT - ceny
    oz = zT - cenz
    orient = jnp.where(nx * ox + ny * oy + nz * oz >= 0.0, 1.0, -1.0)
    nx = nx * orient
    ny = ny * orient
    nz = nz * orient
    nrm = jnp.maximum(jnp.sqrt(nx * nx + ny * ny + nz * nz), 1e-6)
    nx = nx / nrm
    ny = ny / nrm
    nz = nz / nrm

    radius = rsum * invk
    cenrad = jnp.sqrt(ox * ox + oy * oy + oz * oz)
    esum = jnp.maximum(lmin + lmid + lmax, 1e-6)
    dom = lmax / esum
    invariants = jnp.concatenate(
        [lmin, lmid, lmax, radius, cenrad, dom], axis=0)

    h = _gelu(_dotT(Wi0_ref[...], invariants) + bi0_ref[...])
    inv_h = _dotT(Wi1_ref[...], h) + bi1_ref[...]
    fT = featT_ref[0]
    h = _gelu(_dotT(Wf0_ref[...], fT) + bf0_ref[...])
    feat_h = _dotT(Wf1_ref[...], h) + bf1_ref[...]
    hcat = jnp.concatenate([inv_h, feat_h], axis=0)
    h = _gelu(_dotT(Ws0_ref[...], hcat) + bs0_ref[...])
    hidden = _dotT(Ws1_ref[...], h) + bs1_ref[...]
    g0 = _dotT(Wg_ref[...], hidden) + bg_ref[...]

    zrow = jnp.zeros((1, R), jnp.float32)
    onesr = jnp.ones((1, R), jnp.float32)
    pd = -(xT * nx + yT * ny + zT * nz)
    outT = jnp.concatenate([
        g0,
        nx, ny, -nz, pd,
        nx, ny, nz, zrow, zrow, zrow,
        xT, yT, zT, onesr,
        zrow], axis=0)
    out_ref[0] = outT


def kernel(coords, features, W_inv0, b_inv0, W_inv1, b_inv1,
           W_feat0, b_feat0, W_feat1, b_feat1,
           W_sh0, b_sh0, W_sh1, b_sh1, W_g, b_g):
    B, N, _ = coords.shape
    R = 512 if N % 512 == 0 else N
    NB = N // R
    coordsT = jnp.transpose(coords, (0, 2, 1))
    featT = jnp.transpose(features, (0, 2, 1))
    eye = jnp.eye(R, dtype=jnp.float32)
    col = lambda b: b.reshape(-1, 1)

    grid = (B, NB)
    full = lambda b, nb: (0, 0)
    outT = pl.pallas_call(
        functools.partial(_body, R=R, N=N),
        grid=grid,
        in_specs=[
            pl.BlockSpec((1, R, 3), lambda b, nb: (b, nb, 0)),
            pl.BlockSpec((1, 3, N), lambda b, nb: (b, 0, 0)),
            pl.BlockSpec((1, 64, R), lambda b, nb: (b, 0, nb)),
            pl.BlockSpec((R, R), full),
            pl.BlockSpec((6, 64), full), pl.BlockSpec((64, 1), full),
            pl.BlockSpec((64, 64), full), pl.BlockSpec((64, 1), full),
            pl.BlockSpec((64, 64), full), pl.BlockSpec((64, 1), full),
            pl.BlockSpec((64, 64), full), pl.BlockSpec((64, 1), full),
            pl.BlockSpec((128, 64), full), pl.BlockSpec((64, 1), full),
            pl.BlockSpec((64, 64), full), pl.BlockSpec((64, 1), full),
            pl.BlockSpec((64, 1), full), pl.BlockSpec((1, 1), full),
        ],
        out_specs=pl.BlockSpec((1, 16, R), lambda b, nb: (b, 0, nb)),
        out_shape=jax.ShapeDtypeStruct((B, 16, N), jnp.float32),
    )(coords, coordsT, featT, eye,
      W_inv0, col(b_inv0), W_inv1, col(b_inv1),
      W_feat0, col(b_feat0), W_feat1, col(b_feat1),
      W_sh0, col(b_sh0), W_sh1, col(b_sh1),
      W_g, col(b_g))
    return jnp.transpose(outT, (0, 2, 1))


# speedup vs baseline: 103.0262x; 1.6488x over previous
import functools

import jax
import jax.numpy as jnp
from jax.experimental import pallas as pl

KNN = 16
BIG = 3.0e38


def _gelu(x):
    return 0.5 * x * (1.0 + jax.lax.erf(x * 0.7071067811865476))


def _dotT(w, x):
    return jax.lax.dot_general(w, x, (((0,), (0,)), ((), ())),
                               preferred_element_type=jnp.float32)


def _transpose_small(x, eye):
    return jax.lax.dot_general(x, eye, (((0,), (0,)), ((), ())),
                               preferred_element_type=jnp.float32,
                               precision=jax.lax.Precision.HIGHEST)


def _jacobi3(Am, R):
    one = jnp.ones((1, R), jnp.float32)
    zero = jnp.zeros((1, R), jnp.float32)
    V = [[one, zero, zero], [zero, one, zero], [zero, zero, one]]
    for _ in range(5):
        for (p, q) in ((0, 1), (0, 2), (1, 2)):
            app = Am[p][p]
            aqq = Am[q][q]
            apq = Am[p][q]
            small = jnp.abs(apq) < 1e-32
            apq_s = jnp.where(small, 1.0, apq)
            tau = (aqq - app) * 0.5 / apq_s
            sgn = jnp.where(tau >= 0.0, 1.0, -1.0)
            t = sgn / (jnp.abs(tau) + jnp.sqrt(1.0 + tau * tau))
            t = jnp.where(small, 0.0, t)
            c = jax.lax.rsqrt(1.0 + t * t)
            s = t * c
            r = 3 - p - q
            arp = Am[r][p]
            arq = Am[r][q]
            Am[p][p] = app - t * apq
            Am[q][q] = aqq + t * apq
            Am[p][q] = zero
            Am[q][p] = zero
            nrp = c * arp - s * arq
            nrq = s * arp + c * arq
            Am[r][p] = nrp
            Am[p][r] = nrp
            Am[r][q] = nrq
            Am[q][r] = nrq
            for i in range(3):
                vp = V[i][p]
                vq = V[i][q]
                V[i][p] = c * vp - s * vq
                V[i][q] = s * vp + c * vq
    return [Am[0][0], Am[1][1], Am[2][2]], V


def _body(coords_ref, coordsT_ref, featT_ref, eye_ref,
          Wi0_ref, bi0_ref, Wi1_ref, bi1_ref,
          Wf0_ref, bf0_ref, Wf1_ref, bf1_ref,
          Ws0_ref, bs0_ref, Ws1_ref, bs1_ref,
          Wg_ref, bg_ref, out_ref, *, R, N):
    nb = pl.program_id(1)
    ct = coordsT_ref[0]
    cr = coords_ref[0]
    xi = cr[:, 0:1]
    yi = cr[:, 1:2]
    zi = cr[:, 2:3]
    xj = ct[0:1, :]
    yj = ct[1:2, :]
    zj = ct[2:3, :]
    dot = jax.lax.dot_general(cr, ct, (((1,), (0,)), ((), ())),
                              preferred_element_type=jnp.float32)
    sqj = xj * xj + yj * yj + zj * zj
    sqi = xi * xi + yi * yi + zi * zi
    d2 = sqi + sqj - 2.0 * dot
    rows = nb * R + jax.lax.broadcasted_iota(jnp.int32, (R, N), 0)
    cols = jax.lax.broadcasted_iota(jnp.int32, (R, N), 1)
    d2 = jnp.where(rows == cols, BIG, d2)

    def step(_, m):
        cand = jnp.where(d2 > m, d2, BIG)
        return jnp.min(cand, axis=1, keepdims=True)

    t = jax.lax.fori_loop(0, KNN, step,
                          jnp.full((R, 1), -BIG, jnp.float32))
    maskb = d2 <= t
    distm = jnp.sqrt(jnp.where(maskb, jnp.maximum(d2, 0.0), 0.0))

    relx = xj - xi
    rely = yj - yi
    relz = zj - zi
    bx = relx.astype(jnp.bfloat16).astype(jnp.float32)
    by = rely.astype(jnp.bfloat16).astype(jnp.float32)
    bz = relz.astype(jnp.bfloat16).astype(jnp.float32)
    zeros = jnp.zeros((R, N), jnp.float32)
    mbx = jnp.where(maskb, bx, zeros)
    mby = jnp.where(maskb, by, zeros)
    mbz = jnp.where(maskb, bz, zeros)
    invk = 1.0 / float(KNN)

    def rsum1(x):
        return jnp.sum(x, axis=1, keepdims=True)

    scal = jnp.concatenate([
        rsum1(mbx * bx), rsum1(mby * by), rsum1(mbz * bz),
        rsum1(mbx * by), rsum1(mbx * bz), rsum1(mby * bz),
        rsum1(distm), jnp.zeros((R, 1), jnp.float32)], axis=1)
    scalT = _transpose_small(scal, eye_ref[...])
    axx = scalT[0:1] * invk
    ayy = scalT[1:2] * invk
    azz = scalT[2:3] * invk
    axy = scalT[3:4] * invk
    axz = scalT[4:5] * invk
    ayz = scalT[5:6] * invk
    rsum = scalT[6:7]

    ciT = coordsT_ref[0, :, pl.ds(nb * R, R)]
    xT = ciT[0:1]
    yT = ciT[1:2]
    zT = ciT[2:3]

    Am = [[axx, axy, axz], [axy, ayy, ayz], [axz, ayz, azz]]
    (e0, e1, e2), V = _jacobi3(Am, R)

    lmin = jnp.minimum(jnp.minimum(e0, e1), e2)
    lmax = jnp.maximum(jnp.maximum(e0, e1), e2)
    lmid = e0 + e1 + e2 - lmin - lmax
    pick0 = jnp.logical_and(e0 <= e1, e0 <= e2)
    pick1 = jnp.logical_and(jnp.logical_not(pick0), e1 <= e2)
    nx = jnp.where(pick0, V[0][0], jnp.where(pick1, V[0][1], V[0][2]))
    ny = jnp.where(pick0, V[1][0], jnp.where(pick1, V[1][1], V[1][2]))
    nz = jnp.where(pick0, V[2][0], jnp.where(pick1, V[2][1], V[2][2]))

    cenx = jnp.mean(ct[0:1, :], axis=1, keepdims=True)
    ceny = jnp.mean(ct[1:2, :], axis=1, keepdims=True)
    cenz = jnp.mean(ct[2:3, :], axis=1, keepdims=True)
    ox = xT - cenx
    oy = yT - ceny
    oz = zT - cenz
    orient = jnp.where(nx * ox + ny * oy + nz * oz >= 0.0, 1.0, -1.0)
    nx = nx * orient
    ny = ny * orient
    nz = nz * orient
    nrm = jnp.maximum(jnp.sqrt(nx * nx + ny * ny + nz * nz), 1e-6)
    nx = nx / nrm
    ny = ny / nrm
    nz = nz / nrm

    radius = rsum * invk
    cenrad = jnp.sqrt(ox * ox + oy * oy + oz * oz)
    esum = jnp.maximum(lmin + lmid + lmax, 1e-6)
    dom = lmax / esum
    invariants = jnp.concatenate(
        [lmin, lmid, lmax, radius, cenrad, dom], axis=0)

    h = _gelu(_dotT(Wi0_ref[...], invariants) + bi0_ref[...])
    inv_h = _dotT(Wi1_ref[...], h) + bi1_ref[...]
    fT = featT_ref[0]
    h = _gelu(_dotT(Wf0_ref[...], fT) + bf0_ref[...])
    feat_h = _dotT(Wf1_ref[...], h) + bf1_ref[...]
    hcat = jnp.concatenate([inv_h, feat_h], axis=0)
    h = _gelu(_dotT(Ws0_ref[...], hcat) + bs0_ref[...])
    hidden = _dotT(Ws1_ref[...], h) + bs1_ref[...]
    g0 = _dotT(Wg_ref[...], hidden) + bg_ref[...]

    zrow = jnp.zeros((1, R), jnp.float32)
    onesr = jnp.ones((1, R), jnp.float32)
    pd = -(xT * nx + yT * ny + zT * nz)
    outT = jnp.concatenate([
        g0,
        nx, ny, -nz, pd,
        nx, ny, nz, zrow, zrow, zrow,
        xT, yT, zT, onesr,
        zrow], axis=0)
    out_ref[0] = outT


def kernel(coords, features, W_inv0, b_inv0, W_inv1, b_inv1,
           W_feat0, b_feat0, W_feat1, b_feat1,
           W_sh0, b_sh0, W_sh1, b_sh1, W_g, b_g):
    B, N, _ = coords.shape
    R = 512 if N % 512 == 0 else N
    NB = N // R
    coordsT = jnp.transpose(coords, (0, 2, 1))
    featT = jnp.transpose(features, (0, 2, 1))
    eye = jnp.eye(R, dtype=jnp.float32)
    col = lambda b: b.reshape(-1, 1)

    grid = (B, NB)
    full = lambda b, nb: (0, 0)
    outT = pl.pallas_call(
        functools.partial(_body, R=R, N=N),
        grid=grid,
        in_specs=[
            pl.BlockSpec((1, R, 3), lambda b, nb: (b, nb, 0)),
            pl.BlockSpec((1, 3, N), lambda b, nb: (b, 0, 0)),
            pl.BlockSpec((1, 64, R), lambda b, nb: (b, 0, nb)),
            pl.BlockSpec((R, R), full),
            pl.BlockSpec((6, 64), full), pl.BlockSpec((64, 1), full),
            pl.BlockSpec((64, 64), full), pl.BlockSpec((64, 1), full),
            pl.BlockSpec((64, 64), full), pl.BlockSpec((64, 1), full),
            pl.BlockSpec((64, 64), full), pl.BlockSpec((64, 1), full),
            pl.BlockSpec((128, 64), full), pl.BlockSpec((64, 1), full),
            pl.BlockSpec((64, 64), full), pl.BlockSpec((64, 1), full),
            pl.BlockSpec((64, 1), full), pl.BlockSpec((1, 1), full),
        ],
        out_specs=pl.BlockSpec((1, 16, R), lambda b, nb: (b, 0, nb)),
        out_shape=jax.ShapeDtypeStruct((B, 16, N), jnp.float32),
    )(coords, coordsT, featT, eye,
      W_inv0, col(b_inv0), W_inv1, col(b_inv1),
      W_feat0, col(b_feat0), W_feat1, col(b_feat1),
      W_sh0, col(b_sh0), W_sh1, col(b_sh1),
      W_g, col(b_g))
    return jnp.transpose(outT, (0, 2, 1))


# speedup vs baseline: 107.8534x; 1.0469x over previous
import functools

import jax
import jax.numpy as jnp
from jax.experimental import pallas as pl

KNN = 16
BIG = 3.0e38


def _gelu(x):
    return 0.5 * x * (1.0 + jax.lax.erf(x * 0.7071067811865476))


def _dotT(w, x):
    return jax.lax.dot_general(w, x, (((0,), (0,)), ((), ())),
                               preferred_element_type=jnp.float32)


def _transpose_small(x, eye):
    return jax.lax.dot_general(x, eye, (((0,), (0,)), ((), ())),
                               preferred_element_type=jnp.float32,
                               precision=jax.lax.Precision.HIGHEST)


def _jacobi3(Am, R):
    one = jnp.ones((1, R), jnp.float32)
    zero = jnp.zeros((1, R), jnp.float32)
    V = [[one, zero, zero], [zero, one, zero], [zero, zero, one]]
    for _ in range(5):
        for (p, q) in ((0, 1), (0, 2), (1, 2)):
            app = Am[p][p]
            aqq = Am[q][q]
            apq = Am[p][q]
            small = jnp.abs(apq) < 1e-32
            apq_s = jnp.where(small, 1.0, apq)
            tau = (aqq - app) * 0.5 / apq_s
            sgn = jnp.where(tau >= 0.0, 1.0, -1.0)
            t = sgn / (jnp.abs(tau) + jnp.sqrt(1.0 + tau * tau))
            t = jnp.where(small, 0.0, t)
            c = jax.lax.rsqrt(1.0 + t * t)
            s = t * c
            r = 3 - p - q
            arp = Am[r][p]
            arq = Am[r][q]
            Am[p][p] = app - t * apq
            Am[q][q] = aqq + t * apq
            Am[p][q] = zero
            Am[q][p] = zero
            nrp = c * arp - s * arq
            nrq = s * arp + c * arq
            Am[r][p] = nrp
            Am[p][r] = nrp
            Am[r][q] = nrq
            Am[q][r] = nrq
            for i in range(3):
                vp = V[i][p]
                vq = V[i][q]
                V[i][p] = c * vp - s * vq
                V[i][q] = s * vp + c * vq
    return [Am[0][0], Am[1][1], Am[2][2]], V


def _body(coords_ref, coordsT_ref, featT_ref, eye_ref,
          Wi0_ref, bi0_ref, Wi1_ref, bi1_ref,
          Wf0_ref, bf0_ref, Wf1_ref, bf1_ref,
          Ws0_ref, bs0_ref, Ws1_ref, bs1_ref,
          Wg_ref, bg_ref, out_ref, *, R, N):
    nb = pl.program_id(1)
    ct = coordsT_ref[0]
    cr = coords_ref[0]
    xi = cr[:, 0:1]
    yi = cr[:, 1:2]
    zi = cr[:, 2:3]
    xj = ct[0:1, :]
    yj = ct[1:2, :]
    zj = ct[2:3, :]
    dot = jax.lax.dot_general(cr, ct, (((1,), (0,)), ((), ())),
                              preferred_element_type=jnp.float32)
    sqj = xj * xj + yj * yj + zj * zj
    sqi = xi * xi + yi * yi + zi * zi
    d2 = sqi + sqj - 2.0 * dot
    rows = nb * R + jax.lax.broadcasted_iota(jnp.int32, (R, N), 0)
    cols = jax.lax.broadcasted_iota(jnp.int32, (R, N), 1)
    d2 = jnp.where(rows == cols, BIG, d2)

    def step(_, m):
        cand = jnp.where(d2 > m, d2, BIG)
        return jnp.min(cand, axis=1, keepdims=True)

    t = jax.lax.fori_loop(0, KNN, step,
                          jnp.full((R, 1), -BIG, jnp.float32))
    maskb = d2 <= t
    distm = jnp.sqrt(jnp.where(maskb, jnp.maximum(d2, 0.0), 0.0))

    relx = xj - xi
    rely = yj - yi
    relz = zj - zi
    bx = relx.astype(jnp.bfloat16).astype(jnp.float32)
    by = rely.astype(jnp.bfloat16).astype(jnp.float32)
    bz = relz.astype(jnp.bfloat16).astype(jnp.float32)
    zeros = jnp.zeros((R, N), jnp.float32)
    mbx = jnp.where(maskb, bx, zeros)
    mby = jnp.where(maskb, by, zeros)
    mbz = jnp.where(maskb, bz, zeros)
    invk = 1.0 / float(KNN)

    def rsum1(x):
        return jnp.sum(x, axis=1, keepdims=True)

    scal = jnp.concatenate([
        rsum1(mbx * bx), rsum1(mby * by), rsum1(mbz * bz),
        rsum1(mbx * by), rsum1(mbx * bz), rsum1(mby * bz),
        rsum1(distm), jnp.zeros((R, 1), jnp.float32)], axis=1)
    scalT = _transpose_small(scal, eye_ref[...])
    axx = scalT[0:1] * invk
    ayy = scalT[1:2] * invk
    azz = scalT[2:3] * invk
    axy = scalT[3:4] * invk
    axz = scalT[4:5] * invk
    ayz = scalT[5:6] * invk
    rsum = scalT[6:7]

    ciT = coordsT_ref[0, :, pl.ds(nb * R, R)]
    xT = ciT[0:1]
    yT = ciT[1:2]
    zT = ciT[2:3]

    Am = [[axx, axy, axz], [axy, ayy, ayz], [axz, ayz, azz]]
    (e0, e1, e2), V = _jacobi3(Am, R)

    lmin = jnp.minimum(jnp.minimum(e0, e1), e2)
    lmax = jnp.maximum(jnp.maximum(e0, e1), e2)
    lmid = e0 + e1 + e2 - lmin - lmax
    pick0 = jnp.logical_and(e0 <= e1, e0 <= e2)
    pick1 = jnp.logical_and(jnp.logical_not(pick0), e1 <= e2)
    nx = jnp.where(pick0, V[0][0], jnp.where(pick1, V[0][1], V[0][2]))
    ny = jnp.where(pick0, V[1][0], jnp.where(pick1, V[1][1], V[1][2]))
    nz = jnp.where(pick0, V[2][0], jnp.where(pick1, V[2][1], V[2][2]))

    cenx = jnp.mean(ct[0:1, :], axis=1, keepdims=True)
    ceny = jnp.mean(ct[1:2, :], axis=1, keepdims=True)
    cenz = jnp.mean(ct[2:3, :], axis=1, keepdims=True)
    ox = xT - cenx
    oy = yT - ceny
    oz = zT - cenz
    orient = jnp.where(nx * ox + ny * oy + nz * oz >= 0.0, 1.0, -1.0)
    nx = nx * orient
    ny = ny * orient
    nz = nz * orient
    nrm = jnp.maximum(jnp.sqrt(nx * nx + ny * ny + nz * nz), 1e-6)
    nx = nx / nrm
    ny = ny / nrm
    nz = nz / nrm

    radius = rsum * invk
    cenrad = jnp.sqrt(ox * ox + oy * oy + oz * oz)
    esum = jnp.maximum(lmin + lmid + lmax, 1e-6)
    dom = lmax / esum
    invariants = jnp.concatenate(
        [lmin, lmid, lmax, radius, cenrad, dom], axis=0)

    h = _gelu(_dotT(Wi0_ref[...], invariants) + bi0_ref[...])
    inv_h = _dotT(Wi1_ref[...], h) + bi1_ref[...]
    fT = featT_ref[0]
    h = _gelu(_dotT(Wf0_ref[...], fT) + bf0_ref[...])
    feat_h = _dotT(Wf1_ref[...], h) + bf1_ref[...]
    hcat = jnp.concatenate([inv_h, feat_h], axis=0)
    h = _gelu(_dotT(Ws0_ref[...], hcat) + bs0_ref[...])
    hidden = _dotT(Ws1_ref[...], h) + bs1_ref[...]
    g0 = _dotT(Wg_ref[...], hidden) + bg_ref[...]

    zrow = jnp.zeros((1, R), jnp.float32)
    onesr = jnp.ones((1, R), jnp.float32)
    pd = -(xT * nx + yT * ny + zT * nz)
    outT = jnp.concatenate([
        g0,
        nx, ny, -nz, pd,
        nx, ny, nz, zrow, zrow, zrow,
        xT, yT, zT, onesr,
        zrow], axis=0)
    out_ref[0] = outT


def kernel(coords, features, W_inv0, b_inv0, W_inv1, b_inv1,
           W_feat0, b_feat0, W_feat1, b_feat1,
           W_sh0, b_sh0, W_sh1, b_sh1, W_g, b_g):
    B, N, _ = coords.shape
    R = 1024 if N % 1024 == 0 else N
    NB = N // R
    coordsT = jnp.transpose(coords, (0, 2, 1))
    featT = jnp.transpose(features, (0, 2, 1))
    eye = jnp.eye(R, dtype=jnp.float32)
    col = lambda b: b.reshape(-1, 1)

    grid = (B, NB)
    full = lambda b, nb: (0, 0)
    outT = pl.pallas_call(
        functools.partial(_body, R=R, N=N),
        grid=grid,
        in_specs=[
            pl.BlockSpec((1, R, 3), lambda b, nb: (b, nb, 0)),
            pl.BlockSpec((1, 3, N), lambda b, nb: (b, 0, 0)),
            pl.BlockSpec((1, 64, R), lambda b, nb: (b, 0, nb)),
            pl.BlockSpec((R, R), full),
            pl.BlockSpec((6, 64), full), pl.BlockSpec((64, 1), full),
            pl.BlockSpec((64, 64), full), pl.BlockSpec((64, 1), full),
            pl.BlockSpec((64, 64), full), pl.BlockSpec((64, 1), full),
            pl.BlockSpec((64, 64), full), pl.BlockSpec((64, 1), full),
            pl.BlockSpec((128, 64), full), pl.BlockSpec((64, 1), full),
            pl.BlockSpec((64, 64), full), pl.BlockSpec((64, 1), full),
            pl.BlockSpec((64, 1), full), pl.BlockSpec((1, 1), full),
        ],
        out_specs=pl.BlockSpec((1, 16, R), lambda b, nb: (b, 0, nb)),
        out_shape=jax.ShapeDtypeStruct((B, 16, N), jnp.float32),
    )(coords, coordsT, featT, eye,
      W_inv0, col(b_inv0), W_inv1, col(b_inv1),
      W_feat0, col(b_feat0), W_feat1, col(b_feat1),
      W_sh0, col(b_sh0), W_sh1, col(b_sh1),
      W_g, col(b_g))
    return jnp.transpose(outT, (0, 2, 1))
